# baseline (speedup 1.0000x reference)
import functools

import jax
import jax.numpy as jnp
from jax import lax
from jax.experimental import pallas as pl
from jax.experimental.pallas import tpu as pltpu
from jax.experimental.pallas import tpu_sc as plsc

N_PTS = 8192
N_SMP = 16384
W = 16
BQ = 128
CP = 256
N_CORES = 2
N_SUB = 16
SAMP = N_SMP // 2
QC = SAMP // N_CORES
MAP = (0, 1, 2, 3, 1, 4, 5, 6, 2, 5, 7, 8, 3, 6, 8, 9)


def _dist_vals_body(s_ref, n_ref, p_ref, idx_ref, vals_ref):
    s = s_ref[...]
    s0, s1, s2 = s[:, 0:1], s[:, 1:2], s[:, 2:3]
    run_val = jnp.full((BQ, CP), jnp.inf, jnp.float32)
    run_chk = jnp.zeros((BQ, CP), jnp.int32)
    lane = lax.broadcasted_iota(jnp.int32, (BQ, CP), 1)
    for c in range(N_PTS // CP):
        pc = pl.ds(c * CP, CP)
        d2 = (s0 - p_ref[0:1, pc]) ** 2
        d2 = d2 + (s1 - p_ref[1:2, pc]) ** 2
        d2 = d2 + (s2 - p_ref[2:3, pc]) ** 2
        upd = d2 < run_val
        run_val = jnp.where(upd, d2, run_val)
        run_chk = jnp.where(upd, jnp.int32(c), run_chk)
    m = jnp.min(run_val, axis=1, keepdims=True)
    run_idx = run_chk * CP + lane
    cand = jnp.where(run_val == m, run_idx, jnp.int32(0x7FFFFFFF))
    idx_ref[0, 0, :] = jnp.min(cand, axis=1)

    n = n_ref[...]
    d = -((n[:, 0:1] * s0 + n[:, 1:2] * s1) + n[:, 2:3] * s2)
    n3 = n
    ps = jnp.concatenate([n3, d], axis=1)
    one = jnp.ones((BQ, 1), jnp.float32)
    zero2 = jnp.zeros((BQ, 2), jnp.float32)
    a = jnp.concatenate(
        [jnp.broadcast_to(ps[:, 0:1], (BQ, 4)),
         jnp.broadcast_to(ps[:, 1:2], (BQ, 3)),
         jnp.broadcast_to(ps[:, 2:3], (BQ, 2)),
         ps[:, 3:4], n3, one, zero2], axis=1)
    b = jnp.concatenate(
        [ps[:, 0:4], ps[:, 1:4], ps[:, 2:4], ps[:, 3:4],
         jnp.ones((BQ, 4), jnp.float32), zero2], axis=1)
    vals_ref[...] = (a * b).T


def _make_dist_vals(nq=N_SMP, interpret=False):
    return pl.pallas_call(
        _dist_vals_body,
        grid=(nq // BQ,),
        in_specs=[
            pl.BlockSpec((BQ, 3), lambda i: (i, 0)),
            pl.BlockSpec((BQ, 3), lambda i: (i, 0)),
            pl.BlockSpec((8, N_PTS), lambda i: (0, 0)),
        ],
        out_specs=[
            pl.BlockSpec((1, 1, BQ), lambda i: (i, 0, 0)),
            pl.BlockSpec((W, BQ), lambda i: (0, i)),
        ],
        out_shape=[
            jax.ShapeDtypeStruct((nq // BQ, 1, BQ), jnp.int32),
            jax.ShapeDtypeStruct((W, nq), jnp.float32),
        ],
        interpret=interpret,
    )


def _sc_scatter_body(vals_hbm, idx_hbm, zeros_hbm, out_hbm, idx_v, col_v, acc):
    cid = lax.axis_index("c")
    sid = lax.axis_index("s")
    pltpu.sync_copy(zeros_hbm, acc)
    pltpu.sync_copy(idx_hbm.at[pl.ds(cid * QC, QC)], idx_v)
    pltpu.sync_copy(vals_hbm.at[sid, pl.ds(cid * QC, QC)], col_v)

    def body(g, carry):
        iv = idx_v[pl.ds(g * 16, 16)]
        vv = col_v[pl.ds(g * 16, 16)]
        plsc.addupdate_scatter(acc, [iv], vv)
        return carry

    lax.fori_loop(0, QC // 16, body, 0)
    pltpu.sync_copy(acc, out_hbm.at[cid, sid])


@functools.cache
def _make_sc_scatter():
    return functools.partial(
        pl.kernel,
        out_type=jax.ShapeDtypeStruct((N_CORES, W, N_PTS), jnp.float32),
        mesh=plsc.VectorSubcoreMesh(core_axis_name="c", subcore_axis_name="s"),
        compiler_params=pltpu.CompilerParams(needs_layout_passes=False),
        scratch_types=[
            pltpu.VMEM((QC,), jnp.int32),
            pltpu.VMEM((QC,), jnp.float32),
            pltpu.VMEM((N_PTS,), jnp.float32),
        ],
    )(_sc_scatter_body)


def _fin_body(pa_ref, pb_ref, q_ref, m_ref, nv_ref):
    pa = pa_ref[...]
    pb = pb_ref[...]
    tot = (pa[0] + pa[1]) + (pb[0] + pb[1])
    qt = jnp.concatenate([tot[r:r + 1, :] for r in MAP], axis=0)
    q_ref[...] = qt.T
    cnt = tot[13:14, :]
    m_ref[...] = (tot[10:13, :] / jnp.maximum(cnt, 1.0)).T
    nv_ref[...] = (cnt > 0.0).astype(jnp.int32)


def _make_fin(interpret=False):
    return pl.pallas_call(
        _fin_body,
        out_shape=[
            jax.ShapeDtypeStruct((N_PTS, 16), jnp.float32),
            jax.ShapeDtypeStruct((N_PTS, 3), jnp.float32),
            jax.ShapeDtypeStruct((1, N_PTS), jnp.int32),
        ],
        interpret=interpret,
    )


def kernel(samples, normals, points):
    pt = jnp.zeros((8, N_PTS), jnp.float32).at[0:3, :].set(points.T)
    zeros = jnp.zeros((N_PTS,), jnp.float32)
    dist = _make_dist_vals(SAMP)
    idx3a, vals_a = dist(samples[:SAMP], normals[:SAMP], pt)
    part_a = _make_sc_scatter()(vals_a, idx3a.reshape(SAMP), zeros)
    idx3b, vals_b = dist(samples[SAMP:], normals[SAMP:], pt)
    part_b = _make_sc_scatter()(vals_b, idx3b.reshape(SAMP), zeros)
    q, mn, nv_t = _make_fin()(part_a, part_b)
    return (q.reshape(N_PTS, 4, 4), mn, nv_t.reshape(N_PTS).astype(bool))


# speedup vs baseline: 1.2355x; 1.2355x over previous
import functools

import jax
import jax.numpy as jnp
from jax import lax
from jax.experimental import pallas as pl
from jax.experimental.pallas import tpu as pltpu
from jax.experimental.pallas import tpu_sc as plsc

N_PTS = 8192
N_SMP = 16384
W = 16
BQ = 128
CP = 128
N_CORES = 2
N_SUB = 16
SAMP = N_SMP // 2
QC = SAMP // N_CORES
SCQ = 4096
TCQ = N_SMP - SCQ
QT = SCQ // (N_CORES * N_SUB)
MAP = (0, 1, 2, 3, 1, 4, 5, 6, 2, 5, 7, 8, 3, 6, 8, 9)


def _dist_vals_body(s_ref, n_ref, p_ref, idx_ref, vals_ref):
    s = s_ref[...]
    s0, s1, s2 = s[:, 0:1], s[:, 1:2], s[:, 2:3]
    run_val = jnp.full((BQ, CP), jnp.inf, jnp.float32)
    run_chk = jnp.zeros((BQ, CP), jnp.int32)
    lane = lax.broadcasted_iota(jnp.int32, (BQ, CP), 1)
    for c in range(N_PTS // CP):
        pc = pl.ds(c * CP, CP)
        d2 = (s0 - p_ref[0:1, pc]) ** 2
        d2 = d2 + (s1 - p_ref[1:2, pc]) ** 2
        d2 = d2 + (s2 - p_ref[2:3, pc]) ** 2
        upd = d2 < run_val
        run_val = jnp.where(upd, d2, run_val)
        run_chk = jnp.where(upd, jnp.int32(c), run_chk)
    m = jnp.min(run_val, axis=1, keepdims=True)
    run_idx = run_chk * CP + lane
    cand = jnp.where(run_val == m, run_idx, jnp.int32(0x7FFFFFFF))
    idx_ref[0, 0, :] = jnp.min(cand, axis=1)

    n = n_ref[...]
    d = -((n[:, 0:1] * s0 + n[:, 1:2] * s1) + n[:, 2:3] * s2)
    n3 = n
    ps = jnp.concatenate([n3, d], axis=1)
    one = jnp.ones((BQ, 1), jnp.float32)
    zero2 = jnp.zeros((BQ, 2), jnp.float32)
    a = jnp.concatenate(
        [jnp.broadcast_to(ps[:, 0:1], (BQ, 4)),
         jnp.broadcast_to(ps[:, 1:2], (BQ, 3)),
         jnp.broadcast_to(ps[:, 2:3], (BQ, 2)),
         ps[:, 3:4], n3, one, zero2], axis=1)
    b = jnp.concatenate(
        [ps[:, 0:4], ps[:, 1:4], ps[:, 2:4], ps[:, 3:4],
         jnp.ones((BQ, 4), jnp.float32), zero2], axis=1)
    vals_ref[...] = (a * b).T


def _make_dist_vals(nq=N_SMP, interpret=False):
    return pl.pallas_call(
        _dist_vals_body,
        grid=(nq // BQ,),
        in_specs=[
            pl.BlockSpec((BQ, 3), lambda i: (i, 0)),
            pl.BlockSpec((BQ, 3), lambda i: (i, 0)),
            pl.BlockSpec((8, N_PTS), lambda i: (0, 0)),
        ],
        out_specs=[
            pl.BlockSpec((1, 1, BQ), lambda i: (i, 0, 0)),
            pl.BlockSpec((W, BQ), lambda i: (0, i)),
        ],
        out_shape=[
            jax.ShapeDtypeStruct((nq // BQ, 1, BQ), jnp.int32),
            jax.ShapeDtypeStruct((W, nq), jnp.float32),
        ],
        interpret=interpret,
    )


def _sc_scatter_body(v0_hbm, i0_hbm, v1_hbm, i1_hbm, zeros_hbm, out_hbm,
                     idx_v, col_v, acc):
    cid = lax.axis_index("c")
    sid = lax.axis_index("s")
    pltpu.sync_copy(zeros_hbm, acc)

    @pl.when(cid == 0)
    def _():
        pltpu.sync_copy(i0_hbm, idx_v)
        pltpu.sync_copy(v0_hbm.at[sid], col_v)

    @pl.when(cid == 1)
    def _():
        pltpu.sync_copy(i1_hbm, idx_v)
        pltpu.sync_copy(v1_hbm.at[sid], col_v)

    def body(g, carry):
        iv = idx_v[pl.ds(g * 16, 16)]
        vv = col_v[pl.ds(g * 16, 16)]
        plsc.addupdate_scatter(acc, [iv], vv)
        return carry

    lax.fori_loop(0, QC // 16, body, 0)
    pltpu.sync_copy(acc, out_hbm.at[cid, sid])


@functools.cache
def _make_sc_scatter():
    return functools.partial(
        pl.kernel,
        out_type=jax.ShapeDtypeStruct((N_CORES, W, N_PTS), jnp.float32),
        mesh=plsc.VectorSubcoreMesh(core_axis_name="c", subcore_axis_name="s"),
        compiler_params=pltpu.CompilerParams(needs_layout_passes=False),
        scratch_types=[
            pltpu.VMEM((QC,), jnp.int32),
            pltpu.VMEM((QC,), jnp.float32),
            pltpu.VMEM((N_PTS,), jnp.float32),
        ],
    )(_sc_scatter_body)


def _sc_dist_body(pt_hbm, st_hbm, nt_hbm, idx_hbm, vals_hbm,
                  p0_v, p1_v, p2_v, sv_v, nv_v, vals_v, idx_v):
    cid = lax.axis_index("c")
    sid = lax.axis_index("s")
    base = (cid * N_SUB + sid) * QT
    pltpu.sync_copy(pt_hbm.at[0], p0_v)
    pltpu.sync_copy(pt_hbm.at[1], p1_v)
    pltpu.sync_copy(pt_hbm.at[2], p2_v)
    pltpu.sync_copy(st_hbm.at[:, pl.ds(base, QT)], sv_v)
    pltpu.sync_copy(nt_hbm.at[:, pl.ds(base, QT)], nv_v)

    one16 = jnp.ones((16,), jnp.float32)
    zero16 = jnp.zeros((16,), jnp.float32)
    lane16 = lax.iota(jnp.int32, 16)
    big = jnp.int32(0x7FFFFFFF)
    for g in range(QT // 16):
        gs = pl.ds(g * 16, 16)
        n0, n1, n2 = nv_v[0, gs], nv_v[1, gs], nv_v[2, gs]
        s0, s1, s2 = sv_v[0, gs], sv_v[1, gs], sv_v[2, gs]
        d = -((n0 * s0 + n1 * s1) + n2 * s2)
        vals_v[0, gs] = n0 * n0
        vals_v[1, gs] = n0 * n1
        vals_v[2, gs] = n0 * n2
        vals_v[3, gs] = n0 * d
        vals_v[4, gs] = n1 * n1
        vals_v[5, gs] = n1 * n2
        vals_v[6, gs] = n1 * d
        vals_v[7, gs] = n2 * n2
        vals_v[8, gs] = n2 * d
        vals_v[9, gs] = d * d
        vals_v[10, gs] = n0
        vals_v[11, gs] = n1
        vals_v[12, gs] = n2
        vals_v[13, gs] = one16
        vals_v[14, gs] = zero16
        vals_v[15, gs] = zero16

        idxg = jnp.zeros((16,), jnp.int32)
        for h in range(2):
            a0 = [jnp.full((16,), s0[h * 8 + j]) for j in range(8)]
            a1 = [jnp.full((16,), s1[h * 8 + j]) for j in range(8)]
            a2 = [jnp.full((16,), s2[h * 8 + j]) for j in range(8)]
            init = tuple([jnp.full((16,), jnp.inf) for _ in range(8)]
                         + [jnp.zeros((16,), jnp.int32) for _ in range(8)])

            def chunk(c, carry, a0=a0, a1=a1, a2=a2):
                rv, rc = carry[:8], carry[8:]
                cs = pl.ds(c * 16, 16)
                pa, pb, pc = p0_v[cs], p1_v[cs], p2_v[cs]
                cvec = jnp.full((16,), c, jnp.int32)
                nrv, nrc = [], []
                for j in range(8):
                    t0 = a0[j] - pa
                    t1 = a1[j] - pb
                    t2 = a2[j] - pc
                    d2 = (t0 * t0 + t1 * t1) + t2 * t2
                    upd = d2 < rv[j]
                    nrv.append(jnp.where(upd, d2, rv[j]))
                    nrc.append(jnp.where(upd, cvec, rc[j]))
                return tuple(nrv + nrc)

            fin = lax.fori_loop(0, N_PTS // 16, chunk, init)
            rv, rc = fin[:8], fin[8:]
            for j in range(8):
                m = jnp.min(rv[j])
                gidx = rc[j] * 16 + lane16
                cand = jnp.where(rv[j] == m, gidx, big)
                best = jnp.min(cand)
                idxg = jnp.where(lane16 == (h * 8 + j), best, idxg)
        idx_v[gs] = idxg

    pltpu.sync_copy(vals_v, vals_hbm.at[:, pl.ds(base, QT)])
    pltpu.sync_copy(idx_v, idx_hbm.at[pl.ds(base, QT)])


@functools.cache
def _make_sc_dist():
    return functools.partial(
        pl.kernel,
        out_type=[
            jax.ShapeDtypeStruct((SCQ,), jnp.int32),
            jax.ShapeDtypeStruct((W, SCQ), jnp.float32),
        ],
        mesh=plsc.VectorSubcoreMesh(core_axis_name="c", subcore_axis_name="s"),
        compiler_params=pltpu.CompilerParams(needs_layout_passes=False),
        scratch_types=[
            pltpu.VMEM((N_PTS,), jnp.float32),
            pltpu.VMEM((N_PTS,), jnp.float32),
            pltpu.VMEM((N_PTS,), jnp.float32),
            pltpu.VMEM((3, QT), jnp.float32),
            pltpu.VMEM((3, QT), jnp.float32),
            pltpu.VMEM((W, QT), jnp.float32),
            pltpu.VMEM((QT,), jnp.int32),
        ],
    )(_sc_dist_body)


def _fin_body(pa_ref, pb_ref, q_ref, m_ref, nv_ref):
    pa = pa_ref[...]
    pb = pb_ref[...]
    tot = (pa[0] + pa[1]) + (pb[0] + pb[1])
    qt = jnp.concatenate([tot[r:r + 1, :] for r in MAP], axis=0)
    q_ref[...] = qt.T
    cnt = tot[13:14, :]
    m_ref[...] = (tot[10:13, :] / jnp.maximum(cnt, 1.0)).T
    nv_ref[...] = (cnt > 0.0).astype(jnp.int32)


def _make_fin(interpret=False):
    return pl.pallas_call(
        _fin_body,
        out_shape=[
            jax.ShapeDtypeStruct((N_PTS, 16), jnp.float32),
            jax.ShapeDtypeStruct((N_PTS, 3), jnp.float32),
            jax.ShapeDtypeStruct((1, N_PTS), jnp.int32),
        ],
        interpret=interpret,
    )


def kernel(samples, normals, points):
    pt = jnp.zeros((8, N_PTS), jnp.float32).at[0:3, :].set(points.T)
    zeros = jnp.zeros((N_PTS,), jnp.float32)
    st = samples[TCQ:].T
    nt = normals[TCQ:].T
    idx_sc, vals_sc = _make_sc_dist()(pt, st, nt)
    idx3, vals_tc = _make_dist_vals(TCQ)(samples[:TCQ], normals[:TCQ], pt)
    idx_tc = idx3.reshape(TCQ)
    part_a = _make_sc_scatter()(
        vals_tc[:, 0:QC], idx_tc[0:QC],
        vals_tc[:, QC:2 * QC], idx_tc[QC:2 * QC], zeros)
    part_b = _make_sc_scatter()(
        vals_tc[:, 2 * QC:3 * QC], idx_tc[2 * QC:3 * QC],
        vals_sc, idx_sc, zeros)
    q, mn, nv_t = _make_fin()(part_a, part_b)
    return (q.reshape(N_PTS, 4, 4), mn, nv_t.reshape(N_PTS).astype(bool))


# speedup vs baseline: 1.2768x; 1.0334x over previous
import functools

import jax
import jax.numpy as jnp
from jax import lax
from jax.experimental import pallas as pl
from jax.experimental.pallas import tpu as pltpu
from jax.experimental.pallas import tpu_sc as plsc

N_PTS = 8192
N_SMP = 16384
W = 16
BQ = 128
CP = 128
N_CORES = 2
N_SUB = 16
SAMP = N_SMP // 2
QC = SAMP // N_CORES
SCQ = 4096
TCQ = N_SMP - SCQ
QT = SCQ // (N_CORES * N_SUB)
MAP = (0, 1, 2, 3, 1, 4, 5, 6, 2, 5, 7, 8, 3, 6, 8, 9)


def _dist_vals_body(s_ref, n_ref, p_ref, idx_ref, vals_ref):
    s = s_ref[...]
    s0, s1, s2 = s[:, 0:1], s[:, 1:2], s[:, 2:3]
    run_val = jnp.full((BQ, CP), jnp.inf, jnp.float32)
    run_chk = jnp.zeros((BQ, CP), jnp.int32)
    lane = lax.broadcasted_iota(jnp.int32, (BQ, CP), 1)
    for c in range(N_PTS // CP):
        pc = pl.ds(c * CP, CP)
        d2 = (s0 - p_ref[0:1, pc]) ** 2
        d2 = d2 + (s1 - p_ref[1:2, pc]) ** 2
        d2 = d2 + (s2 - p_ref[2:3, pc]) ** 2
        upd = d2 < run_val
        run_val = jnp.where(upd, d2, run_val)
        run_chk = jnp.where(upd, jnp.int32(c), run_chk)
    m = jnp.min(run_val, axis=1, keepdims=True)
    run_idx = run_chk * CP + lane
    cand = jnp.where(run_val == m, run_idx, jnp.int32(0x7FFFFFFF))
    idx_ref[0, 0, :] = jnp.min(cand, axis=1)

    n = n_ref[...]
    d = -((n[:, 0:1] * s0 + n[:, 1:2] * s1) + n[:, 2:3] * s2)
    n3 = n
    ps = jnp.concatenate([n3, d], axis=1)
    one = jnp.ones((BQ, 1), jnp.float32)
    zero2 = jnp.zeros((BQ, 2), jnp.float32)
    a = jnp.concatenate(
        [jnp.broadcast_to(ps[:, 0:1], (BQ, 4)),
         jnp.broadcast_to(ps[:, 1:2], (BQ, 3)),
         jnp.broadcast_to(ps[:, 2:3], (BQ, 2)),
         ps[:, 3:4], n3, one, zero2], axis=1)
    b = jnp.concatenate(
        [ps[:, 0:4], ps[:, 1:4], ps[:, 2:4], ps[:, 3:4],
         jnp.ones((BQ, 4), jnp.float32), zero2], axis=1)
    vals_ref[...] = (a * b).T


def _make_dist_vals(nq=N_SMP, interpret=False):
    return pl.pallas_call(
        _dist_vals_body,
        grid=(nq // BQ,),
        in_specs=[
            pl.BlockSpec((BQ, 3), lambda i: (i, 0)),
            pl.BlockSpec((BQ, 3), lambda i: (i, 0)),
            pl.BlockSpec((8, N_PTS), lambda i: (0, 0)),
        ],
        out_specs=[
            pl.BlockSpec((1, 1, BQ), lambda i: (i, 0, 0)),
            pl.BlockSpec((W, BQ), lambda i: (0, i)),
        ],
        out_shape=[
            jax.ShapeDtypeStruct((nq // BQ, 1, BQ), jnp.int32),
            jax.ShapeDtypeStruct((W, nq), jnp.float32),
        ],
        interpret=interpret,
    )


def _sc_scatter_body(v0_hbm, i0_hbm, v1_hbm, i1_hbm, zeros_hbm, out_hbm,
                     idx_v, col_v, acc):
    cid = lax.axis_index("c")
    sid = lax.axis_index("s")
    pltpu.sync_copy(zeros_hbm, acc)

    @pl.when(cid == 0)
    def _():
        pltpu.sync_copy(i0_hbm, idx_v)
        pltpu.sync_copy(v0_hbm.at[sid], col_v)

    @pl.when(cid == 1)
    def _():
        pltpu.sync_copy(i1_hbm, idx_v)
        pltpu.sync_copy(v1_hbm.at[sid], col_v)

    def body(g, carry):
        iv = idx_v[pl.ds(g * 16, 16)]
        vv = col_v[pl.ds(g * 16, 16)]
        plsc.addupdate_scatter(acc, [iv], vv)
        return carry

    lax.fori_loop(0, QC // 16, body, 0)
    pltpu.sync_copy(acc, out_hbm.at[cid, sid])


@functools.cache
def _make_sc_scatter():
    return functools.partial(
        pl.kernel,
        out_type=jax.ShapeDtypeStruct((N_CORES, W, N_PTS), jnp.float32),
        mesh=plsc.VectorSubcoreMesh(core_axis_name="c", subcore_axis_name="s"),
        compiler_params=pltpu.CompilerParams(needs_layout_passes=False),
        scratch_types=[
            pltpu.VMEM((QC,), jnp.int32),
            pltpu.VMEM((QC,), jnp.float32),
            pltpu.VMEM((N_PTS,), jnp.float32),
        ],
    )(_sc_scatter_body)


def _sc_dist_body(pt_hbm, st_hbm, nt_hbm, idx_hbm, vals_hbm,
                  p0_v, p1_v, p2_v, sv_v, nv_v, vals_v, idx_v):
    cid = lax.axis_index("c")
    sid = lax.axis_index("s")
    base = (cid * N_SUB + sid) * QT
    pltpu.sync_copy(pt_hbm.at[0], p0_v)
    pltpu.sync_copy(pt_hbm.at[1], p1_v)
    pltpu.sync_copy(pt_hbm.at[2], p2_v)
    pltpu.sync_copy(st_hbm.at[:, pl.ds(base, QT)], sv_v)
    pltpu.sync_copy(nt_hbm.at[:, pl.ds(base, QT)], nv_v)

    one16 = jnp.ones((16,), jnp.float32)
    zero16 = jnp.zeros((16,), jnp.float32)
    lane16 = lax.iota(jnp.int32, 16)
    big = jnp.int32(0x7FFFFFFF)
    for g in range(QT // 16):
        gs = pl.ds(g * 16, 16)
        n0, n1, n2 = nv_v[0, gs], nv_v[1, gs], nv_v[2, gs]
        s0, s1, s2 = sv_v[0, gs], sv_v[1, gs], sv_v[2, gs]
        d = -((n0 * s0 + n1 * s1) + n2 * s2)
        vals_v[0, gs] = n0 * n0
        vals_v[1, gs] = n0 * n1
        vals_v[2, gs] = n0 * n2
        vals_v[3, gs] = n0 * d
        vals_v[4, gs] = n1 * n1
        vals_v[5, gs] = n1 * n2
        vals_v[6, gs] = n1 * d
        vals_v[7, gs] = n2 * n2
        vals_v[8, gs] = n2 * d
        vals_v[9, gs] = d * d
        vals_v[10, gs] = n0
        vals_v[11, gs] = n1
        vals_v[12, gs] = n2
        vals_v[13, gs] = one16
        vals_v[14, gs] = zero16
        vals_v[15, gs] = zero16

        idxg = jnp.zeros((16,), jnp.int32)
        for h in range(2):
            a0 = [jnp.full((16,), s0[h * 8 + j]) for j in range(8)]
            a1 = [jnp.full((16,), s1[h * 8 + j]) for j in range(8)]
            a2 = [jnp.full((16,), s2[h * 8 + j]) for j in range(8)]
            init = tuple([jnp.full((16,), jnp.inf) for _ in range(8)]
                         + [jnp.zeros((16,), jnp.int32) for _ in range(8)])

            def chunk(c, carry, a0=a0, a1=a1, a2=a2):
                rv, rc = carry[:8], carry[8:]
                cs = pl.ds(c * 16, 16)
                pa, pb, pc = p0_v[cs], p1_v[cs], p2_v[cs]
                cvec = jnp.full((16,), c, jnp.int32)
                nrv, nrc = [], []
                for j in range(8):
                    t0 = a0[j] - pa
                    t1 = a1[j] - pb
                    t2 = a2[j] - pc
                    d2 = (t0 * t0 + t1 * t1) + t2 * t2
                    upd = d2 < rv[j]
                    nrv.append(jnp.where(upd, d2, rv[j]))
                    nrc.append(jnp.where(upd, cvec, rc[j]))
                return tuple(nrv + nrc)

            fin = lax.fori_loop(0, N_PTS // 16, chunk, init)
            rv, rc = fin[:8], fin[8:]
            for j in range(8):
                m = jnp.min(rv[j])
                gidx = rc[j] * 16 + lane16
                cand = jnp.where(rv[j] == m, gidx, big)
                best = jnp.min(cand)
                idxg = jnp.where(lane16 == (h * 8 + j), best, idxg)
        idx_v[gs] = idxg

    pltpu.sync_copy(vals_v, vals_hbm.at[:, pl.ds(base, QT)])
    pltpu.sync_copy(idx_v, idx_hbm.at[pl.ds(base, QT)])


@functools.cache
def _make_sc_dist():
    return functools.partial(
        pl.kernel,
        out_type=[
            jax.ShapeDtypeStruct((SCQ,), jnp.int32),
            jax.ShapeDtypeStruct((W, SCQ), jnp.float32),
        ],
        mesh=plsc.VectorSubcoreMesh(core_axis_name="c", subcore_axis_name="s"),
        compiler_params=pltpu.CompilerParams(needs_layout_passes=False),
        scratch_types=[
            pltpu.VMEM((N_PTS,), jnp.float32),
            pltpu.VMEM((N_PTS,), jnp.float32),
            pltpu.VMEM((N_PTS,), jnp.float32),
            pltpu.VMEM((3, QT), jnp.float32),
            pltpu.VMEM((3, QT), jnp.float32),
            pltpu.VMEM((W, QT), jnp.float32),
            pltpu.VMEM((QT,), jnp.int32),
        ],
    )(_sc_dist_body)


def _fin_body(pa_ref, pb_ref, q_ref, m_ref, nv_ref):
    pa = pa_ref[...]
    pb = pb_ref[...]
    tot = (pa[0] + pa[1]) + (pb[0] + pb[1])
    qt = jnp.concatenate([tot[r:r + 1, :] for r in MAP], axis=0)
    q_ref[...] = qt.T
    cnt = tot[13:14, :]
    m_ref[...] = (tot[10:13, :] / jnp.maximum(cnt, 1.0)).T
    nv_ref[...] = (cnt > 0.0).astype(jnp.int32)


def _make_fin(interpret=False):
    return pl.pallas_call(
        _fin_body,
        out_shape=[
            jax.ShapeDtypeStruct((N_PTS, 16), jnp.float32),
            jax.ShapeDtypeStruct((N_PTS, 3), jnp.float32),
            jax.ShapeDtypeStruct((1, N_PTS), jnp.int32),
        ],
        interpret=interpret,
    )


def kernel(samples, normals, points):
    pt = jnp.zeros((8, N_PTS), jnp.float32).at[0:3, :].set(points.T)
    zeros = jnp.zeros((N_PTS,), jnp.float32)
    st = samples[TCQ:].T
    nt = normals[TCQ:].T
    idx_sc, vals_sc = _make_sc_dist()(pt, st, nt)
    idx3a, vals_ta = _make_dist_vals(SAMP)(samples[:SAMP], normals[:SAMP], pt)
    idx_ta = idx3a.reshape(SAMP)
    part_a = _make_sc_scatter()(
        vals_ta[:, 0:QC], idx_ta[0:QC],
        vals_ta[:, QC:2 * QC], idx_ta[QC:2 * QC], zeros)
    idx3b, vals_tb = _make_dist_vals(SCQ)(samples[SAMP:TCQ], normals[SAMP:TCQ], pt)
    part_b = _make_sc_scatter()(
        vals_tb, idx3b.reshape(SCQ),
        vals_sc, idx_sc, zeros)
    q, mn, nv_t = _make_fin()(part_a, part_b)
    return (q.reshape(N_PTS, 4, 4), mn, nv_t.reshape(N_PTS).astype(bool))


# speedup vs baseline: 1.2888x; 1.0094x over previous
import functools

import jax
import jax.numpy as jnp
from jax import lax
from jax.experimental import pallas as pl
from jax.experimental.pallas import tpu as pltpu
from jax.experimental.pallas import tpu_sc as plsc

N_PTS = 8192
N_SMP = 16384
W = 16
BQ = 128
CP = 128
N_CORES = 2
N_SUB = 16
SAMP = N_SMP // 2
QC = SAMP // N_CORES
SCQ = 4096
TCQ = N_SMP - SCQ
QT = SCQ // (N_CORES * N_SUB)
MAP = (0, 1, 2, 3, 1, 4, 5, 6, 2, 5, 7, 8, 3, 6, 8, 9)


def _dist_vals_body(s_ref, n_ref, p_ref, idx_ref, vals_ref):
    s = s_ref[...]
    s0, s1, s2 = s[:, 0:1], s[:, 1:2], s[:, 2:3]
    run_val = jnp.full((BQ, CP), jnp.inf, jnp.float32)
    run_chk = jnp.zeros((BQ, CP), jnp.int32)
    lane = lax.broadcasted_iota(jnp.int32, (BQ, CP), 1)
    for c in range(N_PTS // CP):
        pc = pl.ds(c * CP, CP)
        d2 = (s0 - p_ref[0:1, pc]) ** 2
        d2 = d2 + (s1 - p_ref[1:2, pc]) ** 2
        d2 = d2 + (s2 - p_ref[2:3, pc]) ** 2
        upd = d2 < run_val
        run_val = jnp.where(upd, d2, run_val)
        run_chk = jnp.where(upd, jnp.int32(c), run_chk)
    m = jnp.min(run_val, axis=1, keepdims=True)
    run_idx = run_chk * CP + lane
    cand = jnp.where(run_val == m, run_idx, jnp.int32(0x7FFFFFFF))
    idx_ref[0, 0, :] = jnp.min(cand, axis=1)

    n = n_ref[...]
    d = -((n[:, 0:1] * s0 + n[:, 1:2] * s1) + n[:, 2:3] * s2)
    n3 = n
    ps = jnp.concatenate([n3, d], axis=1)
    one = jnp.ones((BQ, 1), jnp.float32)
    zero2 = jnp.zeros((BQ, 2), jnp.float32)
    a = jnp.concatenate(
        [jnp.broadcast_to(ps[:, 0:1], (BQ, 4)),
         jnp.broadcast_to(ps[:, 1:2], (BQ, 3)),
         jnp.broadcast_to(ps[:, 2:3], (BQ, 2)),
         ps[:, 3:4], n3, one, zero2], axis=1)
    b = jnp.concatenate(
        [ps[:, 0:4], ps[:, 1:4], ps[:, 2:4], ps[:, 3:4],
         jnp.ones((BQ, 4), jnp.float32), zero2], axis=1)
    vals_ref[...] = (a * b).T


def _make_dist_vals(nq=N_SMP, interpret=False):
    return pl.pallas_call(
        _dist_vals_body,
        grid=(nq // BQ,),
        in_specs=[
            pl.BlockSpec((BQ, 3), lambda i: (i, 0)),
            pl.BlockSpec((BQ, 3), lambda i: (i, 0)),
            pl.BlockSpec((8, N_PTS), lambda i: (0, 0)),
        ],
        out_specs=[
            pl.BlockSpec((1, 1, BQ), lambda i: (i, 0, 0)),
            pl.BlockSpec((W, BQ), lambda i: (0, i)),
        ],
        out_shape=[
            jax.ShapeDtypeStruct((nq // BQ, 1, BQ), jnp.int32),
            jax.ShapeDtypeStruct((W, nq), jnp.float32),
        ],
        interpret=interpret,
    )


def _sc_scatter_body(v0_hbm, i0_hbm, v1_hbm, i1_hbm, zeros_hbm, out_hbm,
                     idx_v, col_v, acc):
    cid = lax.axis_index("c")
    sid = lax.axis_index("s")
    pltpu.sync_copy(zeros_hbm, acc)

    @pl.when(cid == 0)
    def _():
        pltpu.sync_copy(i0_hbm, idx_v)
        pltpu.sync_copy(v0_hbm.at[sid], col_v)

    @pl.when(cid == 1)
    def _():
        pltpu.sync_copy(i1_hbm, idx_v)
        pltpu.sync_copy(v1_hbm.at[sid], col_v)

    def body(g, carry):
        iv = idx_v[pl.ds(g * 16, 16)]
        vv = col_v[pl.ds(g * 16, 16)]
        plsc.addupdate_scatter(acc, [iv], vv)
        return carry

    lax.fori_loop(0, QC // 16, body, 0)
    pltpu.sync_copy(acc, out_hbm.at[cid, sid])


def _sc_scatter1_body(v_hbm, i_hbm, zeros_hbm, out_hbm, idx_v, col_v, acc):
    cid = lax.axis_index("c")
    sid = lax.axis_index("s")
    pltpu.sync_copy(zeros_hbm, acc)
    pltpu.sync_copy(i_hbm.at[pl.ds(cid * QC, QC)], idx_v)
    pltpu.sync_copy(v_hbm.at[sid, pl.ds(cid * QC, QC)], col_v)

    def body(g, carry):
        iv = idx_v[pl.ds(g * 16, 16)]
        vv = col_v[pl.ds(g * 16, 16)]
        plsc.addupdate_scatter(acc, [iv], vv)
        return carry

    lax.fori_loop(0, QC // 16, body, 0)
    pltpu.sync_copy(acc, out_hbm.at[cid, sid])


@functools.cache
def _make_sc_scatter1():
    return functools.partial(
        pl.kernel,
        out_type=jax.ShapeDtypeStruct((N_CORES, W, N_PTS), jnp.float32),
        mesh=plsc.VectorSubcoreMesh(core_axis_name="c", subcore_axis_name="s"),
        compiler_params=pltpu.CompilerParams(needs_layout_passes=False),
        scratch_types=[
            pltpu.VMEM((QC,), jnp.int32),
            pltpu.VMEM((QC,), jnp.float32),
            pltpu.VMEM((N_PTS,), jnp.float32),
        ],
    )(_sc_scatter1_body)


@functools.cache
def _make_sc_scatter():
    return functools.partial(
        pl.kernel,
        out_type=jax.ShapeDtypeStruct((N_CORES, W, N_PTS), jnp.float32),
        mesh=plsc.VectorSubcoreMesh(core_axis_name="c", subcore_axis_name="s"),
        compiler_params=pltpu.CompilerParams(needs_layout_passes=False),
        scratch_types=[
            pltpu.VMEM((QC,), jnp.int32),
            pltpu.VMEM((QC,), jnp.float32),
            pltpu.VMEM((N_PTS,), jnp.float32),
        ],
    )(_sc_scatter_body)


def _sc_dist_body(pt_hbm, st_hbm, nt_hbm, idx_hbm, vals_hbm,
                  p0_v, p1_v, p2_v, sv_v, nv_v, vals_v, idx_v):
    cid = lax.axis_index("c")
    sid = lax.axis_index("s")
    base = (cid * N_SUB + sid) * QT
    pltpu.sync_copy(pt_hbm.at[0], p0_v)
    pltpu.sync_copy(pt_hbm.at[1], p1_v)
    pltpu.sync_copy(pt_hbm.at[2], p2_v)
    pltpu.sync_copy(st_hbm.at[:, pl.ds(base, QT)], sv_v)
    pltpu.sync_copy(nt_hbm.at[:, pl.ds(base, QT)], nv_v)

    one16 = jnp.ones((16,), jnp.float32)
    zero16 = jnp.zeros((16,), jnp.float32)
    lane16 = lax.iota(jnp.int32, 16)
    big = jnp.int32(0x7FFFFFFF)
    for g in range(QT // 16):
        gs = pl.ds(g * 16, 16)
        n0, n1, n2 = nv_v[0, gs], nv_v[1, gs], nv_v[2, gs]
        s0, s1, s2 = sv_v[0, gs], sv_v[1, gs], sv_v[2, gs]
        d = -((n0 * s0 + n1 * s1) + n2 * s2)
        vals_v[0, gs] = n0 * n0
        vals_v[1, gs] = n0 * n1
        vals_v[2, gs] = n0 * n2
        vals_v[3, gs] = n0 * d
        vals_v[4, gs] = n1 * n1
        vals_v[5, gs] = n1 * n2
        vals_v[6, gs] = n1 * d
        vals_v[7, gs] = n2 * n2
        vals_v[8, gs] = n2 * d
        vals_v[9, gs] = d * d
        vals_v[10, gs] = n0
        vals_v[11, gs] = n1
        vals_v[12, gs] = n2
        vals_v[13, gs] = one16
        vals_v[14, gs] = zero16
        vals_v[15, gs] = zero16

        idxg = jnp.zeros((16,), jnp.int32)
        for h in range(2):
            a0 = [jnp.full((16,), s0[h * 8 + j]) for j in range(8)]
            a1 = [jnp.full((16,), s1[h * 8 + j]) for j in range(8)]
            a2 = [jnp.full((16,), s2[h * 8 + j]) for j in range(8)]
            init = tuple([jnp.full((16,), jnp.inf) for _ in range(8)]
                         + [jnp.zeros((16,), jnp.int32) for _ in range(8)])

            def chunk(c, carry, a0=a0, a1=a1, a2=a2):
                rv, rc = carry[:8], carry[8:]
                cs = pl.ds(c * 16, 16)
                pa, pb, pc = p0_v[cs], p1_v[cs], p2_v[cs]
                cvec = jnp.full((16,), c, jnp.int32)
                nrv, nrc = [], []
                for j in range(8):
                    t0 = a0[j] - pa
                    t1 = a1[j] - pb
                    t2 = a2[j] - pc
                    d2 = (t0 * t0 + t1 * t1) + t2 * t2
                    upd = d2 < rv[j]
                    nrv.append(jnp.where(upd, d2, rv[j]))
                    nrc.append(jnp.where(upd, cvec, rc[j]))
                return tuple(nrv + nrc)

            fin = lax.fori_loop(0, N_PTS // 16, chunk, init)
            rv, rc = fin[:8], fin[8:]
            for j in range(8):
                m = jnp.min(rv[j])
                gidx = rc[j] * 16 + lane16
                cand = jnp.where(rv[j] == m, gidx, big)
                best = jnp.min(cand)
                idxg = jnp.where(lane16 == (h * 8 + j), best, idxg)
        idx_v[gs] = idxg

    pltpu.sync_copy(vals_v, vals_hbm.at[:, pl.ds(base, QT)])
    pltpu.sync_copy(idx_v, idx_hbm.at[pl.ds(base, QT)])


@functools.cache
def _make_sc_dist():
    return functools.partial(
        pl.kernel,
        out_type=[
            jax.ShapeDtypeStruct((SCQ,), jnp.int32),
            jax.ShapeDtypeStruct((W, SCQ), jnp.float32),
        ],
        mesh=plsc.VectorSubcoreMesh(core_axis_name="c", subcore_axis_name="s"),
        compiler_params=pltpu.CompilerParams(needs_layout_passes=False),
        scratch_types=[
            pltpu.VMEM((N_PTS,), jnp.float32),
            pltpu.VMEM((N_PTS,), jnp.float32),
            pltpu.VMEM((N_PTS,), jnp.float32),
            pltpu.VMEM((3, QT), jnp.float32),
            pltpu.VMEM((3, QT), jnp.float32),
            pltpu.VMEM((W, QT), jnp.float32),
            pltpu.VMEM((QT,), jnp.int32),
        ],
    )(_sc_dist_body)


def _fin_body(pa_ref, pb_ref, q_ref, m_ref, nv_ref):
    pa = pa_ref[...]
    pb = pb_ref[...]
    tot = (pa[0] + pa[1]) + (pb[0] + pb[1])
    qt = jnp.concatenate([tot[r:r + 1, :] for r in MAP], axis=0)
    q_ref[...] = qt.T
    cnt = tot[13:14, :]
    m_ref[...] = (tot[10:13, :] / jnp.maximum(cnt, 1.0)).T
    nv_ref[...] = (cnt > 0.0).astype(jnp.int32)


def _make_fin(interpret=False):
    return pl.pallas_call(
        _fin_body,
        out_shape=[
            jax.ShapeDtypeStruct((N_PTS, 16), jnp.float32),
            jax.ShapeDtypeStruct((N_PTS, 3), jnp.float32),
            jax.ShapeDtypeStruct((1, N_PTS), jnp.int32),
        ],
        interpret=interpret,
    )


def kernel(samples, normals, points):
    pt = jnp.zeros((8, N_PTS), jnp.float32).at[0:3, :].set(points.T)
    zeros = jnp.zeros((N_PTS,), jnp.float32)
    st = samples[TCQ:].T
    nt = normals[TCQ:].T
    idx_sc, vals_sc = _make_sc_dist()(pt, st, nt)
    idx3a, vals_ta = _make_dist_vals(SAMP)(samples[:SAMP], normals[:SAMP], pt)
    part_a = _make_sc_scatter1()(vals_ta, idx3a.reshape(SAMP), zeros)
    idx3b, vals_tb = _make_dist_vals(SCQ)(samples[SAMP:TCQ], normals[SAMP:TCQ], pt)
    part_b = _make_sc_scatter()(
        vals_tb, idx3b.reshape(SCQ),
        vals_sc, idx_sc, zeros)
    q, mn, nv_t = _make_fin()(part_a, part_b)
    return (q.reshape(N_PTS, 4, 4), mn, nv_t.reshape(N_PTS).astype(bool))


# speedup vs baseline: 1.3090x; 1.0156x over previous
import functools

import jax
import jax.numpy as jnp
from jax import lax
from jax.experimental import pallas as pl
from jax.experimental.pallas import tpu as pltpu
from jax.experimental.pallas import tpu_sc as plsc

N_PTS = 8192
N_SMP = 16384
W = 16
BQ = 128
CP = 128
N_CORES = 2
N_SUB = 16
SAMP = N_SMP // 2
QC = SAMP // N_CORES
SCQ = 4096
TCQ = N_SMP - SCQ
QT = SCQ // (N_CORES * N_SUB)
MAP = (0, 1, 2, 3, 1, 4, 5, 6, 2, 5, 7, 8, 3, 6, 8, 9)


def _dist_vals_body(s_ref, n_ref, p_ref, idx_ref, vals_ref):
    s = s_ref[...]
    s0, s1, s2 = s[:, 0:1], s[:, 1:2], s[:, 2:3]
    run_val = jnp.full((BQ, CP), jnp.inf, jnp.float32)
    run_chk = jnp.zeros((BQ, CP), jnp.int32)
    lane = lax.broadcasted_iota(jnp.int32, (BQ, CP), 1)
    for c in range(N_PTS // CP):
        pc = pl.ds(c * CP, CP)
        d2 = (s0 - p_ref[0:1, pc]) ** 2
        d2 = d2 + (s1 - p_ref[1:2, pc]) ** 2
        d2 = d2 + (s2 - p_ref[2:3, pc]) ** 2
        run_chk = jnp.where(d2 < run_val, jnp.int32(c), run_chk)
        run_val = jnp.minimum(run_val, d2)
    m = jnp.min(run_val, axis=1, keepdims=True)
    run_idx = run_chk * CP + lane
    cand = jnp.where(run_val == m, run_idx, jnp.int32(0x7FFFFFFF))
    idx_ref[0, 0, :] = jnp.min(cand, axis=1)

    n = n_ref[...]
    d = -((n[:, 0:1] * s0 + n[:, 1:2] * s1) + n[:, 2:3] * s2)
    n3 = n
    ps = jnp.concatenate([n3, d], axis=1)
    one = jnp.ones((BQ, 1), jnp.float32)
    zero2 = jnp.zeros((BQ, 2), jnp.float32)
    a = jnp.concatenate(
        [jnp.broadcast_to(ps[:, 0:1], (BQ, 4)),
         jnp.broadcast_to(ps[:, 1:2], (BQ, 3)),
         jnp.broadcast_to(ps[:, 2:3], (BQ, 2)),
         ps[:, 3:4], n3, one, zero2], axis=1)
    b = jnp.concatenate(
        [ps[:, 0:4], ps[:, 1:4], ps[:, 2:4], ps[:, 3:4],
         jnp.ones((BQ, 4), jnp.float32), zero2], axis=1)
    vals_ref[...] = (a * b).T


def _make_dist_vals(nq=N_SMP, interpret=False):
    return pl.pallas_call(
        _dist_vals_body,
        grid=(nq // BQ,),
        in_specs=[
            pl.BlockSpec((BQ, 3), lambda i: (i, 0)),
            pl.BlockSpec((BQ, 3), lambda i: (i, 0)),
            pl.BlockSpec((8, N_PTS), lambda i: (0, 0)),
        ],
        out_specs=[
            pl.BlockSpec((1, 1, BQ), lambda i: (i, 0, 0)),
            pl.BlockSpec((W, BQ), lambda i: (0, i)),
        ],
        out_shape=[
            jax.ShapeDtypeStruct((nq // BQ, 1, BQ), jnp.int32),
            jax.ShapeDtypeStruct((W, nq), jnp.float32),
        ],
        interpret=interpret,
    )


def _sc_scatter_body(v0_hbm, i0_hbm, v1_hbm, i1_hbm, zeros_hbm, out_hbm,
                     idx_v, col_v, acc):
    cid = lax.axis_index("c")
    sid = lax.axis_index("s")
    pltpu.sync_copy(zeros_hbm, acc)

    @pl.when(cid == 0)
    def _():
        pltpu.sync_copy(i0_hbm, idx_v)
        pltpu.sync_copy(v0_hbm.at[sid], col_v)

    @pl.when(cid == 1)
    def _():
        pltpu.sync_copy(i1_hbm, idx_v)
        pltpu.sync_copy(v1_hbm.at[sid], col_v)

    def body(g, carry):
        iv = idx_v[pl.ds(g * 16, 16)]
        vv = col_v[pl.ds(g * 16, 16)]
        plsc.addupdate_scatter(acc, [iv], vv)
        return carry

    lax.fori_loop(0, QC // 16, body, 0)
    pltpu.sync_copy(acc, out_hbm.at[cid, sid])


def _sc_scatter1_body(v_hbm, i_hbm, zeros_hbm, out_hbm, idx_v, col_v, acc):
    cid = lax.axis_index("c")
    sid = lax.axis_index("s")
    pltpu.sync_copy(zeros_hbm, acc)
    pltpu.sync_copy(i_hbm.at[pl.ds(cid * QC, QC)], idx_v)
    pltpu.sync_copy(v_hbm.at[sid, pl.ds(cid * QC, QC)], col_v)

    def body(g, carry):
        iv = idx_v[pl.ds(g * 16, 16)]
        vv = col_v[pl.ds(g * 16, 16)]
        plsc.addupdate_scatter(acc, [iv], vv)
        return carry

    lax.fori_loop(0, QC // 16, body, 0)
    pltpu.sync_copy(acc, out_hbm.at[cid, sid])


@functools.cache
def _make_sc_scatter1():
    return functools.partial(
        pl.kernel,
        out_type=jax.ShapeDtypeStruct((N_CORES, W, N_PTS), jnp.float32),
        mesh=plsc.VectorSubcoreMesh(core_axis_name="c", subcore_axis_name="s"),
        compiler_params=pltpu.CompilerParams(needs_layout_passes=False),
        scratch_types=[
            pltpu.VMEM((QC,), jnp.int32),
            pltpu.VMEM((QC,), jnp.float32),
            pltpu.VMEM((N_PTS,), jnp.float32),
        ],
    )(_sc_scatter1_body)


@functools.cache
def _make_sc_scatter():
    return functools.partial(
        pl.kernel,
        out_type=jax.ShapeDtypeStruct((N_CORES, W, N_PTS), jnp.float32),
        mesh=plsc.VectorSubcoreMesh(core_axis_name="c", subcore_axis_name="s"),
        compiler_params=pltpu.CompilerParams(needs_layout_passes=False),
        scratch_types=[
            pltpu.VMEM((QC,), jnp.int32),
            pltpu.VMEM((QC,), jnp.float32),
            pltpu.VMEM((N_PTS,), jnp.float32),
        ],
    )(_sc_scatter_body)


def _sc_dist_body(pt_hbm, st_hbm, nt_hbm, idx_hbm, vals_hbm,
                  p0_v, p1_v, p2_v, sv_v, nv_v, vals_v, idx_v):
    cid = lax.axis_index("c")
    sid = lax.axis_index("s")
    base = (cid * N_SUB + sid) * QT
    pltpu.sync_copy(pt_hbm.at[0], p0_v)
    pltpu.sync_copy(pt_hbm.at[1], p1_v)
    pltpu.sync_copy(pt_hbm.at[2], p2_v)
    pltpu.sync_copy(st_hbm.at[:, pl.ds(base, QT)], sv_v)
    pltpu.sync_copy(nt_hbm.at[:, pl.ds(base, QT)], nv_v)

    one16 = jnp.ones((16,), jnp.float32)
    zero16 = jnp.zeros((16,), jnp.float32)
    lane16 = lax.iota(jnp.int32, 16)
    big = jnp.int32(0x7FFFFFFF)
    for g in range(QT // 16):
        gs = pl.ds(g * 16, 16)
        n0, n1, n2 = nv_v[0, gs], nv_v[1, gs], nv_v[2, gs]
        s0, s1, s2 = sv_v[0, gs], sv_v[1, gs], sv_v[2, gs]
        d = -((n0 * s0 + n1 * s1) + n2 * s2)
        vals_v[0, gs] = n0 * n0
        vals_v[1, gs] = n0 * n1
        vals_v[2, gs] = n0 * n2
        vals_v[3, gs] = n0 * d
        vals_v[4, gs] = n1 * n1
        vals_v[5, gs] = n1 * n2
        vals_v[6, gs] = n1 * d
        vals_v[7, gs] = n2 * n2
        vals_v[8, gs] = n2 * d
        vals_v[9, gs] = d * d
        vals_v[10, gs] = n0
        vals_v[11, gs] = n1
        vals_v[12, gs] = n2
        vals_v[13, gs] = one16
        vals_v[14, gs] = zero16
        vals_v[15, gs] = zero16

        idxg = jnp.zeros((16,), jnp.int32)
        for h in range(2):
            a0 = [jnp.full((16,), s0[h * 8 + j]) for j in range(8)]
            a1 = [jnp.full((16,), s1[h * 8 + j]) for j in range(8)]
            a2 = [jnp.full((16,), s2[h * 8 + j]) for j in range(8)]
            init = tuple([jnp.full((16,), jnp.inf) for _ in range(8)]
                         + [jnp.zeros((16,), jnp.int32) for _ in range(8)])

            def chunk(c, carry, a0=a0, a1=a1, a2=a2):
                rv, rc = carry[:8], carry[8:]
                cs = pl.ds(c * 16, 16)
                pa, pb, pc = p0_v[cs], p1_v[cs], p2_v[cs]
                cvec = jnp.full((16,), c, jnp.int32)
                nrv, nrc = [], []
                for j in range(8):
                    t0 = a0[j] - pa
                    t1 = a1[j] - pb
                    t2 = a2[j] - pc
                    d2 = (t0 * t0 + t1 * t1) + t2 * t2
                    nrc.append(jnp.where(d2 < rv[j], cvec, rc[j]))
                    nrv.append(jnp.minimum(rv[j], d2))
                return tuple(nrv + nrc)

            fin = lax.fori_loop(0, N_PTS // 16, chunk, init)
            rv, rc = fin[:8], fin[8:]
            for j in range(8):
                m = jnp.min(rv[j])
                gidx = rc[j] * 16 + lane16
                cand = jnp.where(rv[j] == m, gidx, big)
                best = jnp.min(cand)
                idxg = jnp.where(lane16 == (h * 8 + j), best, idxg)
        idx_v[gs] = idxg

    pltpu.sync_copy(vals_v, vals_hbm.at[:, pl.ds(base, QT)])
    pltpu.sync_copy(idx_v, idx_hbm.at[pl.ds(base, QT)])


@functools.cache
def _make_sc_dist():
    return functools.partial(
        pl.kernel,
        out_type=[
            jax.ShapeDtypeStruct((SCQ,), jnp.int32),
            jax.ShapeDtypeStruct((W, SCQ), jnp.float32),
        ],
        mesh=plsc.VectorSubcoreMesh(core_axis_name="c", subcore_axis_name="s"),
        compiler_params=pltpu.CompilerParams(needs_layout_passes=False),
        scratch_types=[
            pltpu.VMEM((N_PTS,), jnp.float32),
            pltpu.VMEM((N_PTS,), jnp.float32),
            pltpu.VMEM((N_PTS,), jnp.float32),
            pltpu.VMEM((3, QT), jnp.float32),
            pltpu.VMEM((3, QT), jnp.float32),
            pltpu.VMEM((W, QT), jnp.float32),
            pltpu.VMEM((QT,), jnp.int32),
        ],
    )(_sc_dist_body)


def _fin_body(pa_ref, pb_ref, q_ref, m_ref, nv_ref):
    pa = pa_ref[...]
    pb = pb_ref[...]
    tot = (pa[0] + pa[1]) + (pb[0] + pb[1])
    qt = jnp.concatenate([tot[r:r + 1, :] for r in MAP], axis=0)
    q_ref[...] = qt.T
    cnt = tot[13:14, :]
    m_ref[...] = (tot[10:13, :] / jnp.maximum(cnt, 1.0)).T
    nv_ref[...] = (cnt > 0.0).astype(jnp.int32)


def _make_fin(interpret=False):
    return pl.pallas_call(
        _fin_body,
        out_shape=[
            jax.ShapeDtypeStruct((N_PTS, 16), jnp.float32),
            jax.ShapeDtypeStruct((N_PTS, 3), jnp.float32),
            jax.ShapeDtypeStruct((1, N_PTS), jnp.int32),
        ],
        interpret=interpret,
    )


def kernel(samples, normals, points):
    pt = jnp.zeros((8, N_PTS), jnp.float32).at[0:3, :].set(points.T)
    zeros = jnp.zeros((N_PTS,), jnp.float32)
    st = samples[TCQ:].T
    nt = normals[TCQ:].T
    idx_sc, vals_sc = _make_sc_dist()(pt, st, nt)
    idx3a, vals_ta = _make_dist_vals(SAMP)(samples[:SAMP], normals[:SAMP], pt)
    part_a = _make_sc_scatter1()(vals_ta, idx3a.reshape(SAMP), zeros)
    idx3b, vals_tb = _make_dist_vals(SCQ)(samples[SAMP:TCQ], normals[SAMP:TCQ], pt)
    part_b = _make_sc_scatter()(
        vals_tb, idx3b.reshape(SCQ),
        vals_sc, idx_sc, zeros)
    q, mn, nv_t = _make_fin()(part_a, part_b)
    return (q.reshape(N_PTS, 4, 4), mn, nv_t.reshape(N_PTS).astype(bool))
